# Initial kernel scaffold; baseline (speedup 1.0000x reference)
import jax
import jax.numpy as jnp
from jax.experimental import pallas as pl


def kernel(fragments, alphas, ptclds):
    raise NotImplementedError("write your pallas kernel here")


# speedup vs baseline: 59.6982x; 59.6982x over previous
import functools

import jax
import jax.numpy as jnp
from jax import lax
from jax.experimental import pallas as pl
from jax.experimental.pallas import tpu as pltpu
from jax.experimental.pallas import tpu_sc as plsc

N, K, H, W = 4, 16, 384, 384
C, P = 4, 100000
L = 16

_NC, _NS = 2, 16
_NWORK = _NC * _NS
_WPC = _NWORK // C
_ROWS = N * H
_RPW = _ROWS // _WPC
_NCHUNK = W // L


def _sc_body(frag_hbm, alpha_hbm, tbl_hbm, out_hbm,
             tbl_v, frag_v, alpha_v, out_v):
    wid = lax.axis_index("s") * _NC + lax.axis_index("c")
    chan = wid % C
    slot = wid // C

    pltpu.sync_copy(tbl_hbm.at[pl.ds(chan * P, P)], tbl_v)

    def row_body(i, carry):
        r = slot * _RPW + i
        n = r // H
        h = r % H
        pltpu.sync_copy(
            frag_hbm.at[pl.ds(n * K, K), pl.ds(h * W, W)], frag_v)
        pltpu.sync_copy(
            alpha_hbm.at[pl.ds(n * K, K), pl.ds(h * W, W)], alpha_v)

        def col_body(j, carry2):
            t = jnp.ones((L,), jnp.float32)
            acc = jnp.zeros((L,), jnp.float32)
            for k in range(K):
                f = frag_v[k, pl.ds(j * L, L)]
                a = alpha_v[k, pl.ds(j * L, L)]
                g = plsc.load_gather(tbl_v, [f])
                acc = acc + (a * t) * g
                t = t * (1.0 - a)
            out_v[pl.ds(j * L, L)] = acc
            return carry2

        lax.fori_loop(0, _NCHUNK, col_body, 0, unroll=False)
        pltpu.sync_copy(
            out_v, out_hbm.at[pl.ds(((n * C + chan) * H + h) * W, W)])
        return carry

    lax.fori_loop(0, _RPW, row_body, 0, unroll=False)


@jax.jit
def _run(frag2, alpha2, tbl1):
    mesh = plsc.VectorSubcoreMesh(core_axis_name="c", subcore_axis_name="s")
    f = pl.kernel(
        _sc_body,
        out_type=jax.ShapeDtypeStruct((N * C * H * W,), jnp.float32),
        mesh=mesh,
        scratch_types=[
            pltpu.VMEM((P,), jnp.float32),
            pltpu.VMEM((K, W), jnp.int32),
            pltpu.VMEM((K, W), jnp.float32),
            pltpu.VMEM((W,), jnp.float32),
        ],
        compiler_params=pltpu.CompilerParams(needs_layout_passes=False),
    )
    return f(frag2, alpha2, tbl1)


def kernel(fragments, alphas, ptclds):
    frag2 = fragments.astype(jnp.int32).reshape(N * K, H * W)
    alpha2 = alphas.reshape(N * K, H * W)
    tbl1 = ptclds.reshape(C * P)
    out = _run(frag2, alpha2, tbl1)
    return out.reshape(N, C, H, W)


# speedup vs baseline: 248.2741x; 4.1588x over previous
import jax
import jax.numpy as jnp
from jax import lax
from jax.experimental import pallas as pl
from jax.experimental.pallas import tpu as pltpu
from jax.experimental.pallas import tpu_sc as plsc

N, K, H, W = 4, 16, 384, 384
C, P = 4, 100000
L = 16

_NC, _NS = 2, 16
_NWORK = _NC * _NS
_NPAIR = 2
_WPP = _NWORK // _NPAIR
_ROWS = N * H
_RPW = _ROWS // _WPP
_NCHUNK = W // L
_PPAD = 100096


def _sc_body(frag_hbm, alpha_hbm, tbl_hbm, out_hbm,
             tbl_v, frag_v, alpha_v, out_v,
             sem_f0, sem_f1, sem_a0, sem_a1, sem_o0, sem_o1):
    wid = lax.axis_index("s") * _NC + lax.axis_index("c")
    pair = wid % _NPAIR
    slot = wid // _NPAIR
    base = slot * _RPW

    pltpu.sync_copy(tbl_hbm.at[pl.ds(pair * _PPAD, _PPAD)], tbl_v)

    def issue_in(r, b, semf, sema):
        n = r // H
        h = r % H
        pltpu.async_copy(frag_hbm.at[n, :, h, :], frag_v.at[b], semf)
        pltpu.async_copy(alpha_hbm.at[n, :, h, :], alpha_v.at[b], sema)

    def wait_in(b, semf, sema):
        pltpu.make_async_copy(frag_hbm.at[0, :, 0, :],
                              frag_v.at[b], semf).wait()
        pltpu.make_async_copy(alpha_hbm.at[0, :, 0, :],
                              alpha_v.at[b], sema).wait()

    def wait_out(b, semo):
        pltpu.make_async_copy(out_v.at[b, 0], out_hbm.at[0, 0, 0, :],
                              semo).wait()
        pltpu.make_async_copy(out_v.at[b, 1], out_hbm.at[0, 0, 0, :],
                              semo).wait()

    def compute(r, b, semo):
        def col_body(j, carry):
            t = jnp.ones((L,), jnp.float32)
            acc0 = jnp.zeros((L,), jnp.float32)
            acc1 = jnp.zeros((L,), jnp.float32)
            for k in range(K):
                f = frag_v[b, k, pl.ds(j * L, L)]
                a = alpha_v[b, k, pl.ds(j * L, L)]
                g = plsc.load_gather(tbl_v, [f])
                gb = plsc.bitcast(g, jnp.bfloat16)
                c0, c1 = plsc.unpack(gb, format=plsc.PackFormat.INTERLEAVED)
                w = a * t
                t = t - w
                acc0 = acc0 + w * c0
                acc1 = acc1 + w * c1
            out_v[b, 0, pl.ds(j * L, L)] = acc0
            out_v[b, 1, pl.ds(j * L, L)] = acc1
            return carry

        lax.fori_loop(0, _NCHUNK, col_body, 0, unroll=False)
        n = r // H
        h = r % H
        pltpu.async_copy(out_v.at[b, 0], out_hbm.at[n, 2 * pair, h, :], semo)
        pltpu.async_copy(out_v.at[b, 1], out_hbm.at[n, 2 * pair + 1, h, :],
                         semo)

    issue_in(base, 0, sem_f0, sem_a0)

    def pair_body(i2, carry):
        r0 = base + 2 * i2
        issue_in(r0 + 1, 1, sem_f1, sem_a1)
        wait_in(0, sem_f0, sem_a0)

        @pl.when(i2 > 0)
        def _():
            wait_out(0, sem_o0)
        compute(r0, 0, sem_o0)

        @pl.when(i2 < _RPW // 2 - 1)
        def _():
            issue_in(r0 + 2, 0, sem_f0, sem_a0)
        wait_in(1, sem_f1, sem_a1)

        @pl.when(i2 > 0)
        def _():
            wait_out(1, sem_o1)
        compute(r0 + 1, 1, sem_o1)
        return carry

    lax.fori_loop(0, _RPW // 2, pair_body, 0, unroll=False)
    wait_out(0, sem_o0)
    wait_out(1, sem_o1)


@jax.jit
def _run(frag4, alpha4, tblpacked):
    mesh = plsc.VectorSubcoreMesh(core_axis_name="c", subcore_axis_name="s")
    f = pl.kernel(
        _sc_body,
        out_type=jax.ShapeDtypeStruct((N, C, H, W), jnp.float32),
        mesh=mesh,
        scratch_types=[
            pltpu.VMEM((_PPAD,), jnp.int32),
            pltpu.VMEM((2, K, W), jnp.int32),
            pltpu.VMEM((2, K, W), jnp.float32),
            pltpu.VMEM((2, 2, W), jnp.float32),
            pltpu.SemaphoreType.DMA,
            pltpu.SemaphoreType.DMA,
            pltpu.SemaphoreType.DMA,
            pltpu.SemaphoreType.DMA,
            pltpu.SemaphoreType.DMA,
            pltpu.SemaphoreType.DMA,
        ],
        compiler_params=pltpu.CompilerParams(needs_layout_passes=False),
    )
    return f(frag4, alpha4, tblpacked)


def kernel(fragments, alphas, ptclds):
    frag4 = fragments.astype(jnp.int32)
    ptu = jax.lax.bitcast_convert_type(
        ptclds.astype(jnp.bfloat16), jnp.uint16).astype(jnp.uint32)
    packed = ptu[::2] | (ptu[1::2] << 16)
    packed = jnp.pad(packed, ((0, 0), (0, _PPAD - P)))
    tblpacked = jax.lax.bitcast_convert_type(
        packed, jnp.int32).reshape(_NPAIR * _PPAD)
    return _run(frag4, alphas, tblpacked)
